# baseline (speedup 1.0000x reference)
import dataclasses
import functools

import jax
import jax.numpy as jnp
from jax import lax
from jax.experimental import pallas as pl
from jax.experimental.pallas import tpu as pltpu
from jax.experimental.pallas import tpu_sc as plsc

NUM_CORES = 2
NUM_SUBCORES = 16
NUM_TILES = NUM_CORES * NUM_SUBCORES
EDGE_BLK = 128
LANES = 16
F_ROW, F_ATT, F_COL = 0, 1, 2


@functools.partial(jax.jit, static_argnames=("n_pad", "chunks", "feat"))
def _sc_scatter(h_bf, aux4, zeros_tile, *, n_pad, chunks, feat):
    mesh = plsc.VectorSubcoreMesh(core_axis_name="c", subcore_axis_name="s")
    rows_per_tile = n_pad // NUM_SUBCORES

    cp = pltpu.CompilerParams()
    if "needs_layout_passes" in pltpu.CompilerParams.__dataclass_fields__:
        cp = dataclasses.replace(cp, needs_layout_passes=False)

    @functools.partial(
        pl.kernel,
        mesh=mesh,
        compiler_params=cp,
        out_type=jax.ShapeDtypeStruct((NUM_CORES, n_pad, feat), jnp.float32),
        scratch_types=[
            pltpu.VMEM_SHARED((n_pad, feat), jnp.float32),
            pltpu.VMEM((12, EDGE_BLK), jnp.int32),
            pltpu.VMEM((EDGE_BLK, feat), jnp.float32),
            pltpu.VMEM((EDGE_BLK, feat), jnp.float32),
            pltpu.SemaphoreType.DMA,
            pltpu.SemaphoreType.DMA,
            pltpu.SemaphoreType.DMA,
            pltpu.SemaphoreType.DMA,
            pltpu.SemaphoreType.DMA,
            pltpu.SemaphoreType.DMA,
        ],
    )
    def k(h_hbm, aux_hbm, zeros_hbm, out_hbm,
          acc, aux_v, msgs_a, msgs_b, sga, sgb, sx0, sx1, sx2, sx3):
        c = lax.axis_index("c")
        s = lax.axis_index("s")
        wid = c * NUM_SUBCORES + s
        base = s * rows_per_tile
        sx = (sx0, sx1, sx2, sx3)

        pltpu.sync_copy(zeros_hbm, acc.at[pl.ds(base, rows_per_tile)])
        plsc.subcore_barrier()

        def aux_fetch(j, b):
            pltpu.async_copy(aux_hbm.at[wid, j], aux_v.at[pl.ds(b * 3, 3)], sx[b])

        def aux_wait(b):
            pltpu.make_async_copy(
                aux_hbm.at[wid, 0], aux_v.at[pl.ds(b * 3, 3)], sx[b]).wait()

        def gather(b, msgs, sem):
            pltpu.async_copy(h_hbm.at[aux_v.at[b * 3 + F_COL]], msgs, sem)

        def gather_wait(msgs, sem):
            pltpu.make_async_copy(h_hbm.at[aux_v.at[F_COL]], msgs, sem).wait()

        def scale(msgs, b):
            rb = jnp.full((LANES,), b * 3 + F_ATT, jnp.int32)

            @pl.loop(0, EDGE_BLK)
            def _(e):
                ee = jnp.full((LANES,), e, jnp.int32)
                att = plsc.bitcast(
                    plsc.load_gather(aux_v, [rb, ee]), jnp.float32)
                for kk in range(feat // LANES):
                    sl = pl.ds(kk * LANES, LANES)
                    msgs[e, sl] = msgs[e, sl] * att

        def scatter(msgs, b):
            pltpu.sync_copy(msgs, acc.at[aux_v.at[b * 3 + F_ROW]], add=True)

        for b in range(4):
            aux_fetch(b, b)
        aux_wait(0)
        gather(0, msgs_a, sga)
        aux_wait(1)
        gather(1, msgs_b, sgb)

        @pl.loop(0, chunks, step=4)
        def _(j):
            def clamp(d):
                return jnp.minimum(j + d, chunks - 1)

            gather_wait(msgs_a, sga)
            scale(msgs_a, 0)
            scatter(msgs_a, 0)
            aux_fetch(clamp(4), 0)
            aux_wait(2)
            gather(2, msgs_a, sga)
            gather_wait(msgs_b, sgb)
            scale(msgs_b, 1)
            scatter(msgs_b, 1)
            aux_fetch(clamp(5), 1)
            aux_wait(3)
            gather(3, msgs_b, sgb)
            gather_wait(msgs_a, sga)
            scale(msgs_a, 2)
            scatter(msgs_a, 2)
            aux_fetch(clamp(6), 2)
            aux_wait(0)
            gather(0, msgs_a, sga)
            gather_wait(msgs_b, sgb)
            scale(msgs_b, 3)
            scatter(msgs_b, 3)
            aux_fetch(clamp(7), 3)
            aux_wait(1)
            gather(1, msgs_b, sgb)

        gather_wait(msgs_a, sga)
        gather_wait(msgs_b, sgb)
        aux_wait(2)
        aux_wait(3)

        plsc.subcore_barrier()
        pltpu.sync_copy(
            acc.at[pl.ds(base, rows_per_tile)],
            out_hbm.at[c, pl.ds(base, rows_per_tile)],
        )

    return k(h_bf, aux4, zeros_tile)


def _tc_finish(partials, w, n_out):
    feat = partials.shape[2]
    blk = 1000
    nblk = n_out // blk

    def body(p_ref, w_ref, o_ref):
        x = p_ref[0] + p_ref[1]
        o_ref[...] = jnp.dot(x, w_ref[...], preferred_element_type=jnp.float32)

    return pl.pallas_call(
        body,
        out_shape=jax.ShapeDtypeStruct((n_out, feat), jnp.float32),
        grid=(nblk,),
        in_specs=[
            pl.BlockSpec((NUM_CORES, blk, feat), lambda i: (0, i, 0)),
            pl.BlockSpec((feat, feat), lambda i: (0, 0)),
        ],
        out_specs=pl.BlockSpec((blk, feat), lambda i: (i, 0)),
    )(partials, w)


def kernel(h, edge_index, attn, W_T):
    n_nodes, feat = h.shape
    n_edges = attn.shape[0]
    row = edge_index[0].astype(jnp.int32)
    col = edge_index[1].astype(jnp.int32)
    attn = attn.astype(jnp.float32)

    per = NUM_TILES * EDGE_BLK
    chunks = -(-n_edges // per)
    chunks = -(-chunks // 4) * 4
    e_pad = chunks * per
    pad = e_pad - n_edges
    if pad:
        row = jnp.concatenate([row, jnp.zeros((pad,), jnp.int32)])
        col = jnp.concatenate([col, jnp.zeros((pad,), jnp.int32)])
        attn = jnp.concatenate([attn, jnp.zeros((pad,), jnp.float32)])
    aux4 = jnp.stack(
        [row.reshape(NUM_TILES, chunks, EDGE_BLK),
         lax.bitcast_convert_type(attn, jnp.int32).reshape(
             NUM_TILES, chunks, EDGE_BLK),
         col.reshape(NUM_TILES, chunks, EDGE_BLK)],
        axis=2,
    )
    n_pad = -(-n_nodes // 128) * 128
    h_bf = jnp.pad(h, ((0, n_pad - n_nodes), (0, 0)))
    zeros_tile = jnp.zeros((n_pad // NUM_SUBCORES, feat), jnp.float32)

    partials = _sc_scatter(
        h_bf, aux4, zeros_tile,
        n_pad=n_pad, chunks=chunks, feat=feat,
    )
    return _tc_finish(partials, W_T, n_nodes)
